# baseline (speedup 1.0000x reference)
import functools

import jax
import jax.numpy as jnp
from jax.experimental import pallas as pl

N_NODES = 50000
N_DAGS = 512
N_GLOBAL = 32
IN_DIM = 128
H = 64

BK = 2000
N_BLOCKS = N_NODES // BK


def _act(v):
    return jnp.where(v >= 0, v, 0.01 * v)


def _fused_kernel(x_ref, sm_ref, rd_ref,
                  w1_ref, b1_ref, w2_ref, b2_ref, w3_ref, b3_ref,
                  w4_ref, b4_ref, w5_ref, b5_ref, w6_ref, b6_ref,
                  s1_ref, s2_ref):
    k = pl.program_id(0)

    x = x_ref[...]
    sm = sm_ref[...].reshape(N_DAGS, BK)
    s = _act(jnp.dot(x, w1_ref[...], preferred_element_type=jnp.float32)
             + b1_ref[...])
    s = _act(jnp.dot(s, w2_ref[...], preferred_element_type=jnp.float32)
             + b2_ref[...])
    s = _act(jnp.dot(s, w3_ref[...], preferred_element_type=jnp.float32)
             + b3_ref[...])
    part = jnp.dot(sm, s, preferred_element_type=jnp.float32)

    @pl.when(k == 0)
    def _init():
        s1_ref[...] = part

    @pl.when(k != 0)
    def _acc():
        s1_ref[...] += part

    @pl.when(k == N_BLOCKS - 1)
    def _epilogue():
        s1 = s1_ref[...]
        g = _act(jnp.dot(s1, w4_ref[...], preferred_element_type=jnp.float32)
                 + b4_ref[...])
        g = _act(jnp.dot(g, w5_ref[...], preferred_element_type=jnp.float32)
                 + b5_ref[...])
        g = _act(jnp.dot(g, w6_ref[...], preferred_element_type=jnp.float32)
                 + b6_ref[...])
        s2_ref[...] = jnp.dot(rd_ref[...], g,
                              preferred_element_type=jnp.float32)


@functools.partial(jax.jit, donate_argnums=())
def kernel(summ_mats, running_dags_mat, inputs,
           W1, b1, W2, b2, W3, b3, W4, b4, W5, b5, W6, b6):
    full = lambda shape: pl.BlockSpec(shape, lambda k: (0, 0))
    biases = [b.reshape(1, H) for b in (b1, b2, b3, b4, b5, b6)]

    in_specs = [
        pl.BlockSpec((BK, IN_DIM), lambda k: (k, 0)),
        pl.BlockSpec((N_DAGS, 1, 1, BK), lambda k: (0, k, 0, 0)),
        full((N_GLOBAL, N_DAGS)),
        full((IN_DIM, H)), full((1, H)),
        full((H, H)), full((1, H)),
        full((H, H)), full((1, H)),
        full((H, H)), full((1, H)),
        full((H, H)), full((1, H)),
        full((H, H)), full((1, H)),
    ]
    out_specs = [
        full((N_DAGS, H)),
        full((N_GLOBAL, H)),
    ]
    out_shapes = [
        jax.ShapeDtypeStruct((N_DAGS, H), jnp.float32),
        jax.ShapeDtypeStruct((N_GLOBAL, H), jnp.float32),
    ]

    s1, s2 = pl.pallas_call(
        _fused_kernel,
        grid=(N_BLOCKS,),
        in_specs=in_specs,
        out_specs=out_specs,
        out_shape=out_shapes,
    )(inputs, summ_mats.reshape(N_DAGS, N_BLOCKS, 1, BK), running_dags_mat,
      W1, biases[0], W2, biases[1], W3, biases[2],
      W4, biases[3], W5, biases[4], W6, biases[5])
    return (s1, s2)


# speedup vs baseline: 1.8546x; 1.8546x over previous
import jax
import jax.numpy as jnp
from jax.experimental import pallas as pl

N_NODES = 50000
N_DAGS = 512
N_GLOBAL = 32
IN_DIM = 128
H = 64

BK = 1920
N_BLOCKS = N_NODES // BK
N_TAIL = N_NODES - N_BLOCKS * BK


def _act(v):
    return jnp.where(v >= 0, v, 0.01 * v)


def _mlp(x, w1, b1, w2, b2, w3, b3):
    bf = jnp.bfloat16
    s = _act(jnp.dot(x.astype(bf), w1.astype(bf),
                     preferred_element_type=jnp.float32) + b1)
    s = _act(jnp.dot(s.astype(bf), w2.astype(bf),
                     preferred_element_type=jnp.float32) + b2)
    s = _act(jnp.dot(s.astype(bf), w3.astype(bf),
                     preferred_element_type=jnp.float32) + b3)
    return s


def _fused_kernel(x_ref, sm_ref, xt_ref, st_ref, rd_ref,
                  w1_ref, b1_ref, w2_ref, b2_ref, w3_ref, b3_ref,
                  w4_ref, b4_ref, w5_ref, b5_ref, w6_ref, b6_ref,
                  s1_ref, s2_ref):
    k = pl.program_id(0)
    bf = jnp.bfloat16

    s = _mlp(x_ref[...], w1_ref[...], b1_ref[...], w2_ref[...], b2_ref[...],
             w3_ref[...], b3_ref[...])
    part = jnp.dot(sm_ref[...].astype(bf), s.astype(bf),
                   preferred_element_type=jnp.float32)

    @pl.when(k == 0)
    def _init():
        s_t = _mlp(xt_ref[...], w1_ref[...], b1_ref[...], w2_ref[...],
                   b2_ref[...], w3_ref[...], b3_ref[...])
        tail = jnp.dot(st_ref[...].astype(bf), s_t.astype(bf),
                       preferred_element_type=jnp.float32)
        s1_ref[...] = part + tail

    @pl.when(k != 0)
    def _acc():
        s1_ref[...] += part

    @pl.when(k == N_BLOCKS - 1)
    def _epilogue():
        s1 = s1_ref[...]
        g = _act(jnp.dot(s1, w4_ref[...], preferred_element_type=jnp.float32)
                 + b4_ref[...])
        g = _act(jnp.dot(g, w5_ref[...], preferred_element_type=jnp.float32)
                 + b5_ref[...])
        g = _act(jnp.dot(g, w6_ref[...], preferred_element_type=jnp.float32)
                 + b6_ref[...])
        s2_ref[...] = jnp.dot(rd_ref[...], g,
                              preferred_element_type=jnp.float32)


@jax.jit
def kernel(summ_mats, running_dags_mat, inputs,
           W1, b1, W2, b2, W3, b3, W4, b4, W5, b5, W6, b6):
    full = lambda shape: pl.BlockSpec(shape, lambda k: (0, 0))
    biases = [b.reshape(1, H) for b in (b1, b2, b3, b4, b5, b6)]

    in_specs = [
        pl.BlockSpec((BK, IN_DIM), lambda k: (k, 0)),
        pl.BlockSpec((N_DAGS, BK), lambda k: (0, k)),
        full((N_TAIL, IN_DIM)),
        full((N_DAGS, N_TAIL)),
        full((N_GLOBAL, N_DAGS)),
        full((IN_DIM, H)), full((1, H)),
        full((H, H)), full((1, H)),
        full((H, H)), full((1, H)),
        full((H, H)), full((1, H)),
        full((H, H)), full((1, H)),
        full((H, H)), full((1, H)),
    ]
    out_specs = [
        full((N_DAGS, H)),
        full((N_GLOBAL, H)),
    ]
    out_shapes = [
        jax.ShapeDtypeStruct((N_DAGS, H), jnp.float32),
        jax.ShapeDtypeStruct((N_GLOBAL, H), jnp.float32),
    ]

    body = N_BLOCKS * BK
    s1, s2 = pl.pallas_call(
        _fused_kernel,
        grid=(N_BLOCKS,),
        in_specs=in_specs,
        out_specs=out_specs,
        out_shape=out_shapes,
    )(inputs, summ_mats, inputs[body:], summ_mats[:, body:],
      running_dags_mat,
      W1, biases[0], W2, biases[1], W3, biases[2],
      W4, biases[3], W5, biases[4], W6, biases[5])
    return (s1, s2)


# speedup vs baseline: 1.8770x; 1.0121x over previous
import jax
import jax.numpy as jnp
from jax.experimental import pallas as pl

N_NODES = 50000
N_DAGS = 512
N_GLOBAL = 32
IN_DIM = 128
H = 64

BK = 1920
N_BLOCKS = N_NODES // BK
N_TAIL = N_NODES - N_BLOCKS * BK


def _act(v):
    return jnp.maximum(v, 0.01 * v)


def _dot(a, b):
    return jax.lax.dot_general(
        a, b, (((1,), (0,)), ((), ())),
        precision=jax.lax.Precision.DEFAULT,
        preferred_element_type=jnp.float32)


def _mlp(x, w1, b1, w2, b2, w3, b3):
    s = _act(_dot(x, w1) + b1)
    s = _act(_dot(s, w2) + b2)
    s = _act(_dot(s, w3) + b3)
    return s


def _fused_kernel(x_ref, sm_ref, xt_ref, st_ref, rd_ref,
                  w1_ref, b1_ref, w2_ref, b2_ref, w3_ref, b3_ref,
                  w4_ref, b4_ref, w5_ref, b5_ref, w6_ref, b6_ref,
                  s1_ref, s2_ref):
    k = pl.program_id(0)

    s = _mlp(x_ref[...], w1_ref[...], b1_ref[...], w2_ref[...], b2_ref[...],
             w3_ref[...], b3_ref[...])
    part = _dot(sm_ref[...], s)

    @pl.when(k == 0)
    def _init():
        s_t = _mlp(xt_ref[...], w1_ref[...], b1_ref[...], w2_ref[...],
                   b2_ref[...], w3_ref[...], b3_ref[...])
        s1_ref[...] = part + _dot(st_ref[...], s_t)

    @pl.when(k != 0)
    def _acc():
        s1_ref[...] += part

    @pl.when(k == N_BLOCKS - 1)
    def _epilogue():
        s1 = s1_ref[...]
        g = _act(jnp.dot(s1, w4_ref[...], preferred_element_type=jnp.float32)
                 + b4_ref[...])
        g = _act(jnp.dot(g, w5_ref[...], preferred_element_type=jnp.float32)
                 + b5_ref[...])
        g = _act(jnp.dot(g, w6_ref[...], preferred_element_type=jnp.float32)
                 + b6_ref[...])
        s2_ref[...] = jnp.dot(rd_ref[...], g,
                              preferred_element_type=jnp.float32)


@jax.jit
def kernel(summ_mats, running_dags_mat, inputs,
           W1, b1, W2, b2, W3, b3, W4, b4, W5, b5, W6, b6):
    full = lambda shape: pl.BlockSpec(shape, lambda k: (0, 0))
    biases = [b.reshape(1, H) for b in (b1, b2, b3, b4, b5, b6)]

    in_specs = [
        pl.BlockSpec((BK, IN_DIM), lambda k: (k, 0)),
        pl.BlockSpec((N_DAGS, BK), lambda k: (0, k)),
        full((N_TAIL, IN_DIM)),
        full((N_DAGS, N_TAIL)),
        full((N_GLOBAL, N_DAGS)),
        full((IN_DIM, H)), full((1, H)),
        full((H, H)), full((1, H)),
        full((H, H)), full((1, H)),
        full((H, H)), full((1, H)),
        full((H, H)), full((1, H)),
        full((H, H)), full((1, H)),
    ]
    out_specs = [
        full((N_DAGS, H)),
        full((N_GLOBAL, H)),
    ]
    out_shapes = [
        jax.ShapeDtypeStruct((N_DAGS, H), jnp.float32),
        jax.ShapeDtypeStruct((N_GLOBAL, H), jnp.float32),
    ]

    body = N_BLOCKS * BK
    s1, s2 = pl.pallas_call(
        _fused_kernel,
        grid=(N_BLOCKS,),
        in_specs=in_specs,
        out_specs=out_specs,
        out_shape=out_shapes,
    )(inputs, summ_mats, inputs[body:], summ_mats[:, body:],
      running_dags_mat,
      W1, biases[0], W2, biases[1], W3, biases[2],
      W4, biases[3], W5, biases[4], W6, biases[5])
    return (s1, s2)


# speedup vs baseline: 1.9219x; 1.0239x over previous
import jax
import jax.numpy as jnp
from jax.experimental import pallas as pl

N_NODES = 50000
N_DAGS = 512
N_GLOBAL = 32
IN_DIM = 128
H = 64

BK = 1920
N_BLOCKS = -(-N_NODES // BK)
N_TAIL = N_NODES - (N_BLOCKS - 1) * BK


def _act(v):
    return jnp.maximum(v, 0.01 * v)


def _dot(a, b):
    return jax.lax.dot_general(
        a, b, (((1,), (0,)), ((), ())),
        precision=jax.lax.Precision.DEFAULT,
        preferred_element_type=jnp.float32)


def _mlp(x, w1, b1, w2, b2, w3, b3):
    s = _act(_dot(x, w1) + b1)
    s = _act(_dot(s, w2) + b2)
    s = _act(_dot(s, w3) + b3)
    return s


def _fused_kernel(x_ref, sm_ref, rd_ref,
                  w1_ref, b1_ref, w2_ref, b2_ref, w3_ref, b3_ref,
                  w4_ref, b4_ref, w5_ref, b5_ref, w6_ref, b6_ref,
                  s1_ref, s2_ref):
    k = pl.program_id(0)

    s = _mlp(x_ref[...], w1_ref[...], b1_ref[...], w2_ref[...], b2_ref[...],
             w3_ref[...], b3_ref[...])

    @pl.when(k == N_BLOCKS - 1)
    def _mask_tail():
        rows = jax.lax.broadcasted_iota(jnp.int32, (BK, H), 0)
        s_masked = jnp.where(rows < N_TAIL, s, 0.0)
        cols = jax.lax.broadcasted_iota(jnp.int32, (N_DAGS, BK), 1)
        sm_masked = jnp.where(cols < N_TAIL, sm_ref[...], 0.0)
        s1_ref[...] += _dot(sm_masked, s_masked)

    @pl.when(k == 0)
    def _init():
        s1_ref[...] = _dot(sm_ref[...], s)

    @pl.when(jnp.logical_and(k != 0, k != N_BLOCKS - 1))
    def _acc():
        s1_ref[...] += _dot(sm_ref[...], s)

    @pl.when(k == N_BLOCKS - 1)
    def _epilogue():
        s1 = s1_ref[...]
        g = _act(jnp.dot(s1, w4_ref[...], preferred_element_type=jnp.float32)
                 + b4_ref[...])
        g = _act(jnp.dot(g, w5_ref[...], preferred_element_type=jnp.float32)
                 + b5_ref[...])
        g = _act(jnp.dot(g, w6_ref[...], preferred_element_type=jnp.float32)
                 + b6_ref[...])
        s2_ref[...] = jnp.dot(rd_ref[...], g,
                              preferred_element_type=jnp.float32)


@jax.jit
def kernel(summ_mats, running_dags_mat, inputs,
           W1, b1, W2, b2, W3, b3, W4, b4, W5, b5, W6, b6):
    full = lambda shape: pl.BlockSpec(shape, lambda k: (0, 0))
    biases = [b.reshape(1, H) for b in (b1, b2, b3, b4, b5, b6)]

    in_specs = [
        pl.BlockSpec((BK, IN_DIM), lambda k: (k, 0)),
        pl.BlockSpec((N_DAGS, BK), lambda k: (0, k)),
        full((N_GLOBAL, N_DAGS)),
        full((IN_DIM, H)), full((1, H)),
        full((H, H)), full((1, H)),
        full((H, H)), full((1, H)),
        full((H, H)), full((1, H)),
        full((H, H)), full((1, H)),
        full((H, H)), full((1, H)),
    ]
    out_specs = [
        full((N_DAGS, H)),
        full((N_GLOBAL, H)),
    ]
    out_shapes = [
        jax.ShapeDtypeStruct((N_DAGS, H), jnp.float32),
        jax.ShapeDtypeStruct((N_GLOBAL, H), jnp.float32),
    ]

    s1, s2 = pl.pallas_call(
        _fused_kernel,
        grid=(N_BLOCKS,),
        in_specs=in_specs,
        out_specs=out_specs,
        out_shape=out_shapes,
    )(inputs, summ_mats, running_dags_mat,
      W1, biases[0], W2, biases[1], W3, biases[2],
      W4, biases[3], W5, biases[4], W6, biases[5])
    return (s1, s2)


# speedup vs baseline: 4.7552x; 2.4742x over previous
import jax
import jax.numpy as jnp
from jax.experimental import pallas as pl

N_NODES = 50000
N_DAGS = 512
N_GLOBAL = 32
IN_DIM = 128
H = 64

BK = 2000
N_BLOCKS = N_NODES // BK


def _act(v):
    return jnp.maximum(v, 0.01 * v)


def _dot(a, b, dims):
    return jax.lax.dot_general(
        a, b, (dims, ((), ())),
        precision=jax.lax.Precision.DEFAULT,
        preferred_element_type=jnp.float32)


def _fused_kernel(x_ref, smt_ref, rd_ref,
                  w1t_ref, b1_ref, w2_ref, b2_ref, w3_ref, b3_ref,
                  w4_ref, b4_ref, w5_ref, b5_ref, w6_ref, b6_ref,
                  s1_ref, s2_ref):
    k = pl.program_id(0)

    x = x_ref[...]
    s = _act(_dot(x, w1t_ref[...], ((1,), (1,))) + b1_ref[...])
    s = _act(_dot(s, w2_ref[...], ((1,), (0,))) + b2_ref[...])
    s = _act(_dot(s, w3_ref[...], ((1,), (0,))) + b3_ref[...])
    part = _dot(smt_ref[...], s, ((0,), (0,)))

    @pl.when(k == 0)
    def _init():
        s1_ref[...] = part

    @pl.when(k != 0)
    def _acc():
        s1_ref[...] += part

    @pl.when(k == N_BLOCKS - 1)
    def _epilogue():
        s1 = s1_ref[...]
        g = _act(_dot(s1, w4_ref[...], ((1,), (0,))) + b4_ref[...])
        g = _act(_dot(g, w5_ref[...], ((1,), (0,))) + b5_ref[...])
        g = _act(_dot(g, w6_ref[...], ((1,), (0,))) + b6_ref[...])
        s2_ref[...] = _dot(rd_ref[...], g, ((1,), (0,)))


@jax.jit
def kernel(summ_mats, running_dags_mat, inputs,
           W1, b1, W2, b2, W3, b3, W4, b4, W5, b5, W6, b6):
    full = lambda shape: pl.BlockSpec(shape, lambda k: (0, 0))
    biases = [b.reshape(1, H) for b in (b1, b2, b3, b4, b5, b6)]

    in_specs = [
        pl.BlockSpec((BK, IN_DIM), lambda k: (k, 0)),
        pl.BlockSpec((BK, N_DAGS), lambda k: (k, 0)),
        full((N_GLOBAL, N_DAGS)),
        full((H, IN_DIM)), full((1, H)),
        full((H, H)), full((1, H)),
        full((H, H)), full((1, H)),
        full((H, H)), full((1, H)),
        full((H, H)), full((1, H)),
        full((H, H)), full((1, H)),
    ]
    out_specs = [
        full((N_DAGS, H)),
        full((N_GLOBAL, H)),
    ]
    out_shapes = [
        jax.ShapeDtypeStruct((N_DAGS, H), jnp.float32),
        jax.ShapeDtypeStruct((N_GLOBAL, H), jnp.float32),
    ]

    s1, s2 = pl.pallas_call(
        _fused_kernel,
        grid=(N_BLOCKS,),
        in_specs=in_specs,
        out_specs=out_specs,
        out_shape=out_shapes,
    )(inputs, summ_mats.T, running_dags_mat,
      W1.T, biases[0], W2, biases[1], W3, biases[2],
      W4, biases[3], W5, biases[4], W6, biases[5])
    return (s1, s2)


# speedup vs baseline: 5.2280x; 1.0994x over previous
import jax
import jax.numpy as jnp
from jax.experimental import pallas as pl

N_NODES = 50000
N_DAGS = 512
N_GLOBAL = 32
IN_DIM = 128
H = 64

BK = 5000
N_BLOCKS = N_NODES // BK
N_SUB = 5
SUB = BK // N_SUB


def _act(v):
    return jnp.maximum(v, 0.01 * v)


def _dot(a, b, dims):
    return jax.lax.dot_general(
        a, b, (dims, ((), ())),
        precision=jax.lax.Precision.DEFAULT,
        preferred_element_type=jnp.float32)


def _fused_kernel(x_ref, smt_ref, rd_ref,
                  w1t_ref, b1_ref, w2_ref, b2_ref, w3_ref, b3_ref,
                  w4_ref, b4_ref, w5_ref, b5_ref, w6_ref, b6_ref,
                  s1_ref, s2_ref):
    k = pl.program_id(0)

    parts = []
    for c in range(N_SUB):
        x = x_ref[pl.ds(c * SUB, SUB), :]
        s = _act(_dot(x, w1t_ref[...], ((1,), (1,))) + b1_ref[...])
        s = _act(_dot(s, w2_ref[...], ((1,), (0,))) + b2_ref[...])
        s = _act(_dot(s, w3_ref[...], ((1,), (0,))) + b3_ref[...])
        parts.append(_dot(smt_ref[pl.ds(c * SUB, SUB), :], s, ((0,), (0,))))
    part = sum(parts)

    @pl.when(k == 0)
    def _init():
        s1_ref[...] = part

    @pl.when(k != 0)
    def _acc():
        s1_ref[...] += part

    @pl.when(k == N_BLOCKS - 1)
    def _epilogue():
        s1 = s1_ref[...]
        g = _act(_dot(s1, w4_ref[...], ((1,), (0,))) + b4_ref[...])
        g = _act(_dot(g, w5_ref[...], ((1,), (0,))) + b5_ref[...])
        g = _act(_dot(g, w6_ref[...], ((1,), (0,))) + b6_ref[...])
        s2_ref[...] = _dot(rd_ref[...], g, ((1,), (0,)))


@jax.jit
def kernel(summ_mats, running_dags_mat, inputs,
           W1, b1, W2, b2, W3, b3, W4, b4, W5, b5, W6, b6):
    full = lambda shape: pl.BlockSpec(shape, lambda k: (0, 0))
    biases = [b.reshape(1, H) for b in (b1, b2, b3, b4, b5, b6)]

    in_specs = [
        pl.BlockSpec((BK, IN_DIM), lambda k: (k, 0)),
        pl.BlockSpec((BK, N_DAGS), lambda k: (k, 0)),
        full((N_GLOBAL, N_DAGS)),
        full((H, IN_DIM)), full((1, H)),
        full((H, H)), full((1, H)),
        full((H, H)), full((1, H)),
        full((H, H)), full((1, H)),
        full((H, H)), full((1, H)),
        full((H, H)), full((1, H)),
    ]
    out_specs = [
        full((N_DAGS, H)),
        full((N_GLOBAL, H)),
    ]
    out_shapes = [
        jax.ShapeDtypeStruct((N_DAGS, H), jnp.float32),
        jax.ShapeDtypeStruct((N_GLOBAL, H), jnp.float32),
    ]

    s1, s2 = pl.pallas_call(
        _fused_kernel,
        grid=(N_BLOCKS,),
        in_specs=in_specs,
        out_specs=out_specs,
        out_shape=out_shapes,
    )(inputs, summ_mats.T, running_dags_mat,
      W1.T, biases[0], W2, biases[1], W3, biases[2],
      W4, biases[3], W5, biases[4], W6, biases[5])
    return (s1, s2)


# speedup vs baseline: 5.8703x; 1.1229x over previous
import jax
import jax.numpy as jnp
from jax.experimental import pallas as pl

N_NODES = 50000
N_DAGS = 512
N_GLOBAL = 32
IN_DIM = 128
H = 64

BK = 5000
N_BLOCKS = N_NODES // BK
N_SUB = 5
SUB = BK // N_SUB


def _act(v):
    return jnp.maximum(v, 0.01 * v)


def _dot(a, b, dims):
    return jax.lax.dot_general(
        a, b, (dims, ((), ())),
        precision=jax.lax.Precision.DEFAULT,
        preferred_element_type=jnp.float32)


def _fused_kernel(x_ref, smt_ref, rd_ref,
                  w1t_ref, b1_ref, w2_ref, b2_ref, w3_ref, b3_ref,
                  w4_ref, b4_ref, w5_ref, b5_ref, w6_ref, b6_ref,
                  s1_ref, s2_ref):
    k = pl.program_id(0)

    parts = []
    for c in range(N_SUB):
        x = x_ref[pl.ds(c * SUB, SUB), :]
        s = _act(_dot(x, w1t_ref[...], ((1,), (1,))) + b1_ref[...])
        s = _act(_dot(s, w2_ref[...], ((1,), (0,))) + b2_ref[...])
        s = _act(_dot(s, w3_ref[...], ((1,), (0,))) + b3_ref[...])
        parts.append(_dot(s, smt_ref[pl.ds(c * SUB, SUB), :], ((0,), (0,))))
    part = sum(parts)

    @pl.when(k == 0)
    def _init():
        s1_ref[...] = part

    @pl.when(k != 0)
    def _acc():
        s1_ref[...] += part

    @pl.when(k == N_BLOCKS - 1)
    def _epilogue():
        s1t = s1_ref[...]
        g = _act(_dot(s1t, w4_ref[...], ((0,), (0,))) + b4_ref[...])
        g = _act(_dot(g, w5_ref[...], ((1,), (0,))) + b5_ref[...])
        g = _act(_dot(g, w6_ref[...], ((1,), (0,))) + b6_ref[...])
        s2_ref[...] = _dot(rd_ref[...], g, ((1,), (0,)))


@jax.jit
def kernel(summ_mats, running_dags_mat, inputs,
           W1, b1, W2, b2, W3, b3, W4, b4, W5, b5, W6, b6):
    full = lambda shape: pl.BlockSpec(shape, lambda k: (0, 0))
    biases = [b.reshape(1, H) for b in (b1, b2, b3, b4, b5, b6)]

    in_specs = [
        pl.BlockSpec((BK, IN_DIM), lambda k: (k, 0)),
        pl.BlockSpec((BK, N_DAGS), lambda k: (k, 0)),
        full((N_GLOBAL, N_DAGS)),
        full((H, IN_DIM)), full((1, H)),
        full((H, H)), full((1, H)),
        full((H, H)), full((1, H)),
        full((H, H)), full((1, H)),
        full((H, H)), full((1, H)),
        full((H, H)), full((1, H)),
    ]
    out_specs = [
        full((H, N_DAGS)),
        full((N_GLOBAL, H)),
    ]
    out_shapes = [
        jax.ShapeDtypeStruct((H, N_DAGS), jnp.float32),
        jax.ShapeDtypeStruct((N_GLOBAL, H), jnp.float32),
    ]

    s1t, s2 = pl.pallas_call(
        _fused_kernel,
        grid=(N_BLOCKS,),
        in_specs=in_specs,
        out_specs=out_specs,
        out_shape=out_shapes,
    )(inputs, summ_mats.T, running_dags_mat,
      W1.T, biases[0], W2, biases[1], W3, biases[2],
      W4, biases[3], W5, biases[4], W6, biases[5])
    return (s1t.T, s2)
